# Initial kernel scaffold; baseline (speedup 1.0000x reference)
import jax
import jax.numpy as jnp
from jax.experimental import pallas as pl


def kernel(x, table):
    raise NotImplementedError("write your pallas kernel here")


# speedup vs baseline: 1.6189x; 1.6189x over previous
import functools

import numpy as np
import jax
import jax.numpy as jnp
from jax import lax
from jax.experimental import pallas as pl
from jax.experimental.pallas import tpu as pltpu
from jax.experimental.pallas import tpu_sc as plsc

SEQ = 4096
DIM = 128
_LANES = 16
_NUM_CORES = 2
_NUM_SUBCORES = 16
_NW = _NUM_CORES * _NUM_SUBCORES
_B_PER_W = SEQ // _NW


def _pe_table() -> np.ndarray:
    pos = np.arange(1, SEQ + 1, dtype=np.float64)[:, None]
    i = np.arange(1, DIM + 1, dtype=np.float64)[None, :]
    w_even = (1.0 / 10000.0) ** (i / DIM)
    w_odd = (1.0 / 10000.0) ** ((i - 1.0) / DIM)
    even = (np.arange(1, DIM + 1) % 2 == 0)[None, :]
    return np.where(even, np.sin(pos * w_even), np.cos(pos * w_odd)).astype(
        np.float32
    )


_PE_NP = _pe_table()

_mesh = plsc.VectorSubcoreMesh(core_axis_name="c", subcore_axis_name="s")


@functools.partial(
    pl.kernel,
    mesh=_mesh,
    out_type=jax.ShapeDtypeStruct((SEQ, DIM), jnp.float32),
    scratch_types=[
        pltpu.VMEM((_B_PER_W,), jnp.int32),
        pltpu.VMEM((_B_PER_W, DIM), jnp.float32),
        pltpu.VMEM((_B_PER_W, DIM), jnp.float32),
        pltpu.SemaphoreType.DMA,
        pltpu.SemaphoreType.DMA,
    ],
)
def _emb_pe_kernel(x_hbm, table_hbm, pe_hbm, out_hbm, idx_v, rows_v, pe_v,
                   sem_g, sem_p):
    wid = lax.axis_index("s") * _NUM_CORES + lax.axis_index("c")
    base = wid * _B_PER_W

    pltpu.sync_copy(x_hbm.at[pl.ds(base, _B_PER_W)], idx_v)
    gather = pltpu.async_copy(table_hbm.at[idx_v], rows_v, sem_g)
    pe_cp = pltpu.async_copy(pe_hbm.at[pl.ds(base, _B_PER_W)], pe_v, sem_p)
    gather.wait()
    pe_cp.wait()

    def add_row(i, _):
        for j in range(DIM // _LANES):
            sl = pl.ds(j * _LANES, _LANES)
            plsc.addupdate(rows_v.at[i, sl], pe_v[i, sl])
        return ()

    lax.fori_loop(0, _B_PER_W, add_row, ())

    pltpu.sync_copy(rows_v, out_hbm.at[pl.ds(base, _B_PER_W)])


def kernel(x, table):
    pe = jnp.asarray(_PE_NP)
    return _emb_pe_kernel(x.astype(jnp.int32), table, pe)
